# baseline (speedup 1.0000x reference)
import functools

import jax
import jax.numpy as jnp
from jax import lax
from jax.experimental import pallas as pl
from jax.experimental.pallas import tpu as pltpu
from jax.experimental.pallas import tpu_sc as plsc

VOCAB = 50368
HIDDEN = 768
EPS = 1e-05

N_TOKENS = 4 * 8192
NUM_CORES = 2
NUM_SUBCORES = 16
NUM_WORKERS = NUM_CORES * NUM_SUBCORES
PER_WORKER = N_TOKENS // NUM_WORKERS
CHUNK = 64
NBUF = 2
NUM_CHUNKS = PER_WORKER // CHUNK
LANES = 16
NVEC = HIDDEN // LANES


def _lane_sum(v):
    iota = lax.iota(jnp.int32, LANES)
    dnums = lax.GatherDimensionNumbers(
        offset_dims=(), collapsed_slice_dims=(0,), start_index_map=(0,))
    for sh in (1, 2, 4, 8):
        perm = lax.gather(v, (iota ^ sh)[:, None], dnums, slice_sizes=(1,),
                          mode=lax.GatherScatterMode.PROMISE_IN_BOUNDS)
        v = v + perm
    return v


def _rsqrt(y):
    i = plsc.bitcast(y, jnp.int32)
    i = 0x5F3759DF - (i >> 1)
    g = plsc.bitcast(i, jnp.float32)
    g = g * (1.5 - 0.5 * y * g * g)
    g = g * (1.5 - 0.5 * y * g * g)
    g = g * (1.5 - 0.5 * y * g * g)
    return g


def _ln_row(buf, w_v, r):
    acc = jnp.zeros((LANES,), jnp.float32)
    acc2 = jnp.zeros((LANES,), jnp.float32)
    for j in range(NVEC):
        x = buf[r, pl.ds(LANES * j, LANES)]
        acc = acc + x
        acc2 = acc2 + x * x
    s = _lane_sum(acc)
    s2 = _lane_sum(acc2)
    mean = s * (1.0 / HIDDEN)
    var = s2 * (1.0 / HIDDEN) - mean * mean
    g = _rsqrt(var + EPS)
    a = g
    b = -mean * g
    for j in range(NVEC):
        sl = pl.ds(LANES * j, LANES)
        x = buf[r, sl]
        buf[r, sl] = (x * a + b) * w_v[sl]


def _body(ids_hbm, table_hbm, w_hbm, out_hbm,
          idx_all, rows_v, w_v, gsem0, gsem1, wsem0, wsem1):
    wid = lax.axis_index("s") * NUM_CORES + lax.axis_index("c")
    base = wid * PER_WORKER
    gsems = (gsem0, gsem1)
    wsems = (wsem0, wsem1)

    pltpu.sync_copy(w_hbm, w_v)
    pltpu.sync_copy(ids_hbm.at[pl.ds(base, PER_WORKER)], idx_all)

    def idx_slice(ci):
        return idx_all.at[pl.ds(pl.multiple_of(ci * CHUNK, CHUNK), CHUNK)]

    def out_slice(ci):
        return out_hbm.at[pl.ds(pl.multiple_of(base + ci * CHUNK, CHUNK), CHUNK)]

    def g_start(ci, b):
        pltpu.async_copy(table_hbm.at[idx_slice(ci)], rows_v.at[b], gsems[b])

    def g_wait(ci, b):
        pltpu.make_async_copy(table_hbm.at[idx_slice(ci)], rows_v.at[b],
                              gsems[b]).wait()

    def wb_start(ci, b):
        pltpu.async_copy(rows_v.at[b], out_slice(ci), wsems[b])

    def wb_wait(ci, b):
        pltpu.make_async_copy(rows_v.at[b], out_slice(ci), wsems[b]).wait()

    def compute_chunk(b):
        buf = rows_v.at[b]

        def row_body(r, c):
            r0 = r * 2
            _ln_row(buf, w_v, r0)
            _ln_row(buf, w_v, r0 + 1)
            return c

        lax.fori_loop(0, CHUNK // 2, row_body, 0)

    g_start(0, 0)

    def outer(g, carry):
        for b in range(NBUF):
            ci = g * NBUF + b
            nb = 1 - b
            g_wait(ci, b)

            @pl.when(ci + 1 < NUM_CHUNKS)
            def _():
                @pl.when(ci >= 1)
                def _():
                    wb_wait(ci, nb)
                g_start(ci + 1, nb)

            compute_chunk(b)
            wb_start(ci, b)
        return carry

    lax.fori_loop(0, NUM_CHUNKS // NBUF, outer, 0)

    wb_wait(NUM_CHUNKS - 2, 0)
    wb_wait(NUM_CHUNKS - 1, 1)


_sc_call = functools.partial(
    pl.kernel,
    mesh=plsc.VectorSubcoreMesh(core_axis_name="c", subcore_axis_name="s"),
    out_type=jax.ShapeDtypeStruct((N_TOKENS, HIDDEN), jnp.float32),
    scratch_types=[
        pltpu.VMEM((PER_WORKER,), jnp.int32),
        pltpu.VMEM((NBUF, CHUNK, HIDDEN), jnp.float32),
        pltpu.VMEM((HIDDEN,), jnp.float32),
        pltpu.SemaphoreType.DMA,
        pltpu.SemaphoreType.DMA,
        pltpu.SemaphoreType.DMA,
        pltpu.SemaphoreType.DMA,
    ],
    compiler_params=pltpu.CompilerParams(needs_layout_passes=False),
)(_body)


@jax.jit
def kernel(input_ids, tok_embeddings, norm_weight):
    ids = input_ids.reshape(-1).astype(jnp.int32)
    out = _sc_call(ids, tok_embeddings, norm_weight)
    return out.reshape(input_ids.shape + (HIDDEN,))


# speedup vs baseline: 2.1569x; 2.1569x over previous
import functools

import jax
import jax.numpy as jnp
from jax import lax
from jax.experimental import pallas as pl
from jax.experimental.pallas import tpu as pltpu
from jax.experimental.pallas import tpu_sc as plsc

VOCAB = 50368
HIDDEN = 768
EPS = 1e-05

N_TOKENS = 4 * 8192
NUM_CORES = 2
NUM_SUBCORES = 16
NUM_WORKERS = NUM_CORES * NUM_SUBCORES
PER_WORKER = N_TOKENS // NUM_WORKERS
CHUNK = 64
NBUF = 2
NUM_CHUNKS = PER_WORKER // CHUNK

ROW_BLK = 512


def _gather_body(ids_hbm, table_hbm, out_hbm, idx_all, buf_v, gsem0, gsem1,
                 wsem0, wsem1):
    wid = lax.axis_index("s") * NUM_CORES + lax.axis_index("c")
    base = wid * PER_WORKER
    gsems = (gsem0, gsem1)
    wsems = (wsem0, wsem1)

    pltpu.sync_copy(ids_hbm.at[pl.ds(base, PER_WORKER)], idx_all)

    def idx_slice(ci):
        return idx_all.at[pl.ds(pl.multiple_of(ci * CHUNK, CHUNK), CHUNK)]

    def out_slice(ci):
        return out_hbm.at[pl.ds(pl.multiple_of(base + ci * CHUNK, CHUNK), CHUNK)]

    def g_start(ci, b):
        pltpu.async_copy(table_hbm.at[idx_slice(ci)], buf_v.at[b], gsems[b])

    def g_wait(ci, b):
        pltpu.make_async_copy(table_hbm.at[idx_slice(ci)], buf_v.at[b],
                              gsems[b]).wait()

    def wb_start(ci, b):
        pltpu.async_copy(buf_v.at[b], out_slice(ci), wsems[b])

    def wb_wait(ci, b):
        pltpu.make_async_copy(buf_v.at[b], out_slice(ci), wsems[b]).wait()

    g_start(0, 0)

    def outer(g, carry):
        for b in range(NBUF):
            ci = g * NBUF + b
            nb = 1 - b
            g_wait(ci, b)
            wb_start(ci, b)

            @pl.when(ci + 1 < NUM_CHUNKS)
            def _():
                @pl.when(ci >= 1)
                def _():
                    wb_wait(ci - 1, nb)
                g_start(ci + 1, nb)
        return carry

    lax.fori_loop(0, NUM_CHUNKS // NBUF, outer, 0)

    wb_wait(NUM_CHUNKS - 2, 0)
    wb_wait(NUM_CHUNKS - 1, 1)


_sc_gather = functools.partial(
    pl.kernel,
    mesh=plsc.VectorSubcoreMesh(core_axis_name="c", subcore_axis_name="s"),
    out_type=jax.ShapeDtypeStruct((N_TOKENS, HIDDEN), jnp.float32),
    scratch_types=[
        pltpu.VMEM((PER_WORKER,), jnp.int32),
        pltpu.VMEM((NBUF, CHUNK, HIDDEN), jnp.float32),
        pltpu.SemaphoreType.DMA,
        pltpu.SemaphoreType.DMA,
        pltpu.SemaphoreType.DMA,
        pltpu.SemaphoreType.DMA,
    ],
    compiler_params=pltpu.CompilerParams(needs_layout_passes=False),
)(_gather_body)


def _ln_body(x_ref, w_ref, o_ref):
    x = x_ref[...]
    mean = jnp.mean(x, axis=1, keepdims=True)
    xc = x - mean
    var = jnp.mean(xc * xc, axis=1, keepdims=True)
    o_ref[...] = xc * lax.rsqrt(var + EPS) * w_ref[...]


_tc_layernorm = pl.pallas_call(
    _ln_body,
    grid=(N_TOKENS // ROW_BLK,),
    in_specs=[
        pl.BlockSpec((ROW_BLK, HIDDEN), lambda i: (i, 0)),
        pl.BlockSpec((1, HIDDEN), lambda i: (0, 0)),
    ],
    out_specs=pl.BlockSpec((ROW_BLK, HIDDEN), lambda i: (i, 0)),
    out_shape=jax.ShapeDtypeStruct((N_TOKENS, HIDDEN), jnp.float32),
    compiler_params=pltpu.CompilerParams(
        dimension_semantics=("arbitrary",)),
)


@jax.jit
def kernel(input_ids, tok_embeddings, norm_weight):
    ids = input_ids.reshape(-1).astype(jnp.int32)
    emb = _sc_gather(ids, tok_embeddings)
    out = _tc_layernorm(emb, norm_weight.reshape(1, HIDDEN))
    return out.reshape(input_ids.shape + (HIDDEN,))


# speedup vs baseline: 2.4925x; 1.1556x over previous
import functools

import jax
import jax.numpy as jnp
from jax import lax
from jax.experimental import pallas as pl
from jax.experimental.pallas import tpu as pltpu
from jax.experimental.pallas import tpu_sc as plsc

VOCAB = 50368
HIDDEN = 768
EPS = 1e-05

N_TOKENS = 4 * 8192
NUM_CORES = 2
NUM_SUBCORES = 16
NUM_WORKERS = NUM_CORES * NUM_SUBCORES
PER_WORKER = N_TOKENS // NUM_WORKERS
CHUNK = 64
NBUF = 2
NUM_CHUNKS = PER_WORKER // CHUNK

ROW_BLK = 2048


def _gather_body(ids_hbm, table_hbm, out_hbm, idx_all, buf_v, gsem0, gsem1,
                 wsem0, wsem1):
    wid = lax.axis_index("s") * NUM_CORES + lax.axis_index("c")
    base = wid * PER_WORKER
    gsems = (gsem0, gsem1)
    wsems = (wsem0, wsem1)

    pltpu.sync_copy(ids_hbm.at[pl.ds(base, PER_WORKER)], idx_all)

    def idx_slice(ci):
        return idx_all.at[pl.ds(pl.multiple_of(ci * CHUNK, CHUNK), CHUNK)]

    def out_slice(ci):
        return out_hbm.at[pl.ds(pl.multiple_of(base + ci * CHUNK, CHUNK), CHUNK)]

    def g_start(ci, b):
        pltpu.async_copy(table_hbm.at[idx_slice(ci)], buf_v.at[b], gsems[b])

    def g_wait(ci, b):
        pltpu.make_async_copy(table_hbm.at[idx_slice(ci)], buf_v.at[b],
                              gsems[b]).wait()

    def wb_start(ci, b):
        pltpu.async_copy(buf_v.at[b], out_slice(ci), wsems[b])

    def wb_wait(ci, b):
        pltpu.make_async_copy(buf_v.at[b], out_slice(ci), wsems[b]).wait()

    g_start(0, 0)

    def outer(g, carry):
        for b in range(NBUF):
            ci = g * NBUF + b
            nb = 1 - b
            g_wait(ci, b)
            wb_start(ci, b)

            @pl.when(ci + 1 < NUM_CHUNKS)
            def _():
                @pl.when(ci >= 1)
                def _():
                    wb_wait(ci - 1, nb)
                g_start(ci + 1, nb)
        return carry

    lax.fori_loop(0, NUM_CHUNKS // NBUF, outer, 0)

    wb_wait(NUM_CHUNKS - 2, 0)
    wb_wait(NUM_CHUNKS - 1, 1)


_sc_gather = functools.partial(
    pl.kernel,
    mesh=plsc.VectorSubcoreMesh(core_axis_name="c", subcore_axis_name="s"),
    out_type=jax.ShapeDtypeStruct((N_TOKENS, HIDDEN), jnp.float32),
    scratch_types=[
        pltpu.VMEM((PER_WORKER,), jnp.int32),
        pltpu.VMEM((NBUF, CHUNK, HIDDEN), jnp.float32),
        pltpu.SemaphoreType.DMA,
        pltpu.SemaphoreType.DMA,
        pltpu.SemaphoreType.DMA,
        pltpu.SemaphoreType.DMA,
    ],
    compiler_params=pltpu.CompilerParams(needs_layout_passes=False),
)(_gather_body)


def _ln_body(x_ref, w_ref, o_ref):
    x = x_ref[...]
    mean = jnp.mean(x, axis=1, keepdims=True)
    xc = x - mean
    var = jnp.mean(xc * xc, axis=1, keepdims=True)
    o_ref[...] = xc * lax.rsqrt(var + EPS) * w_ref[...]


_tc_layernorm = pl.pallas_call(
    _ln_body,
    grid=(N_TOKENS // ROW_BLK,),
    in_specs=[
        pl.BlockSpec((ROW_BLK, HIDDEN), lambda i: (i, 0)),
        pl.BlockSpec((1, HIDDEN), lambda i: (0, 0)),
    ],
    out_specs=pl.BlockSpec((ROW_BLK, HIDDEN), lambda i: (i, 0)),
    out_shape=jax.ShapeDtypeStruct((N_TOKENS, HIDDEN), jnp.float32),
    compiler_params=pltpu.CompilerParams(
        dimension_semantics=("arbitrary",)),
)


@jax.jit
def kernel(input_ids, tok_embeddings, norm_weight):
    ids = input_ids.reshape(-1).astype(jnp.int32)
    emb = _sc_gather(ids, tok_embeddings)
    out = _tc_layernorm(emb, norm_weight.reshape(1, HIDDEN))
    return out.reshape(input_ids.shape + (HIDDEN,))
